# baseline (speedup 1.0000x reference)
import functools

import jax
import jax.numpy as jnp
from jax import lax
from jax.experimental import pallas as pl
from jax.experimental.pallas import tpu as pltpu
from jax.experimental.pallas import tpu_sc as plsc

N = 10000
E = 320000
DF = 128
H = 64
G = 64
P = 0.3

NC = 2
NS = 16
NW = NC * NS

CH = 128
EPT = E // NW
KB = 10
KCH = 80
KBCH = KB * CH
EPT_PAD = KCH * CH
NJ = EPT_PAD // KBCH
ACC_ROWS = 10240
ZROWS = ACC_ROWS // NS
PAD_DST = N

_mesh = plsc.VectorSubcoreMesh(core_axis_name="c", subcore_axis_name="s")
_sc_params = pltpu.CompilerParams(use_tc_tiling_on_sc=False)


@functools.partial(
    pl.kernel,
    out_type=jax.ShapeDtypeStruct((NC, ACC_ROWS, 16), jnp.float32),
    mesh=_mesh,
    scratch_types=[
        pltpu.VMEM((NJ, KBCH), jnp.int32),
        pltpu.VMEM((KBCH, 16), jnp.float32),
        pltpu.VMEM_SHARED((ACC_ROWS, 16), jnp.float32),
    ],
    compiler_params=_sc_params,
)
def _sc_deg(dst_hbm, ones_hbm, zeros16_hbm, out_hbm, idx_v, ones_v, acc):
    c = lax.axis_index("c")
    s = lax.axis_index("s")
    wid = c * NS + s
    pltpu.sync_copy(zeros16_hbm, acc.at[pl.ds(s * ZROWS, ZROWS)])
    pltpu.sync_copy(ones_hbm, ones_v)
    pltpu.sync_copy(dst_hbm.at[wid], idx_v)
    plsc.subcore_barrier()

    for j in range(NJ):
        pltpu.sync_copy(ones_v, acc.at[idx_v.at[j]], add=True)
    plsc.subcore_barrier()
    pltpu.sync_copy(acc.at[pl.ds(s * ZROWS, ZROWS)],
                    out_hbm.at[c, pl.ds(s * ZROWS, ZROWS)])


@functools.partial(
    pl.kernel,
    out_type=jax.ShapeDtypeStruct((NC, ACC_ROWS, H), jnp.bfloat16),
    mesh=_mesh,
    scratch_types=[
        pltpu.VMEM((NJ, KBCH), jnp.int32),
        pltpu.VMEM((NJ, KBCH), jnp.int32),
        pltpu.VMEM((2, KBCH, H), jnp.bfloat16),
        pltpu.VMEM_SHARED((ACC_ROWS, H), jnp.bfloat16),
        pltpu.SemaphoreType.DMA((2,)),
        pltpu.SemaphoreType.DMA((2,)),
    ],
    compiler_params=_sc_params,
)
def _sc_edge(hp_hbm, src_hbm, dst_hbm, zeros_hbm, out_hbm,
             sidx_v, didx_v, rows_v, acc, gsem, ssem):
    c = lax.axis_index("c")
    s = lax.axis_index("s")
    wid = c * NS + s
    pltpu.sync_copy(zeros_hbm, acc.at[pl.ds(s * ZROWS, ZROWS)])
    pltpu.sync_copy(src_hbm.at[wid], sidx_v)
    pltpu.sync_copy(dst_hbm.at[wid], didx_v)
    plsc.subcore_barrier()

    pltpu.async_copy(hp_hbm.at[sidx_v.at[0]], rows_v.at[0], gsem.at[0])
    for j in range(NJ):
        b = j % 2
        pltpu.make_async_copy(hp_hbm.at[pl.ds(0, KBCH)], rows_v.at[b],
                              gsem.at[b]).wait()
        if j + 1 < NJ:
            if j >= 1:
                pltpu.make_async_copy(rows_v.at[1 - b], acc.at[pl.ds(0, KBCH)],
                                      ssem.at[1 - b]).wait()
            pltpu.async_copy(hp_hbm.at[sidx_v.at[j + 1]], rows_v.at[1 - b],
                             gsem.at[1 - b])
        pltpu.async_copy(rows_v.at[b], acc.at[didx_v.at[j]], ssem.at[b],
                         add=True)
    pltpu.make_async_copy(rows_v.at[0], acc.at[pl.ds(0, KBCH)],
                          ssem.at[0]).wait()
    pltpu.make_async_copy(rows_v.at[1], acc.at[pl.ds(0, KBCH)],
                          ssem.at[1]).wait()
    plsc.subcore_barrier()
    pltpu.sync_copy(acc.at[pl.ds(s * ZROWS, ZROWS)],
                    out_hbm.at[c, pl.ds(s * ZROWS, ZROWS)])


_BR = 1000


def _tc_pre_body(x_ref, w_ref, d0_ref, d1_ref, hp_ref, hpb_ref, dinv_ref):
    deg = d0_ref[0, :, 0] + d1_ref[0, :, 0] + 1.0
    dinv = lax.rsqrt(deg)
    h = jnp.dot(x_ref[...], w_ref[...], preferred_element_type=jnp.float32)
    hp = h * dinv[:, None]
    hp_ref[...] = hp
    hpb_ref[...] = hp.astype(jnp.bfloat16)
    dinv_ref[...] = dinv[:, None]


def _tc_pre(x, w1, deg_out):
    return pl.pallas_call(
        _tc_pre_body,
        grid=(N // _BR,),
        in_specs=[
            pl.BlockSpec((_BR, DF), lambda i: (i, 0)),
            pl.BlockSpec((DF, H), lambda i: (0, 0)),
            pl.BlockSpec((1, _BR, 16), lambda i: (0, i, 0)),
            pl.BlockSpec((1, _BR, 16), lambda i: (1, i, 0)),
        ],
        out_specs=[
            pl.BlockSpec((_BR, H), lambda i: (i, 0)),
            pl.BlockSpec((_BR, H), lambda i: (i, 0)),
            pl.BlockSpec((_BR, 1), lambda i: (i, 0)),
        ],
        out_shape=[
            jax.ShapeDtypeStruct((N, H), jnp.float32),
            jax.ShapeDtypeStruct((N, H), jnp.bfloat16),
            jax.ShapeDtypeStruct((N, 1), jnp.float32),
        ],
    )(x, w1, deg_out, deg_out)


def _tc_epi_body(a0_ref, a1_ref, hp_ref, dinv_ref, b_ref, m_ref, w_ref,
                 hpn_ref, hpnb_ref):
    dinv = dinv_ref[...]
    a = a0_ref[0].astype(jnp.float32) + a1_ref[0].astype(jnp.float32)
    h = (a + hp_ref[...]) * dinv + b_ref[...]
    h = jnp.maximum(h, 0.0) * m_ref[...]
    hpn = jnp.dot(h, w_ref[...], preferred_element_type=jnp.float32) * dinv
    hpn_ref[...] = hpn
    hpnb_ref[...] = hpn.astype(jnp.bfloat16)


def _tc_epi(acc, hp, dinv, b, m, wn):
    return pl.pallas_call(
        _tc_epi_body,
        grid=(N // _BR,),
        in_specs=[
            pl.BlockSpec((1, _BR, H), lambda i: (0, i, 0)),
            pl.BlockSpec((1, _BR, H), lambda i: (1, i, 0)),
            pl.BlockSpec((_BR, H), lambda i: (i, 0)),
            pl.BlockSpec((_BR, 1), lambda i: (i, 0)),
            pl.BlockSpec((1, H), lambda i: (0, 0)),
            pl.BlockSpec((_BR, H), lambda i: (i, 0)),
            pl.BlockSpec((H, H), lambda i: (0, 0)),
        ],
        out_specs=[
            pl.BlockSpec((_BR, H), lambda i: (i, 0)),
            pl.BlockSpec((_BR, H), lambda i: (i, 0)),
        ],
        out_shape=[
            jax.ShapeDtypeStruct((N, H), jnp.float32),
            jax.ShapeDtypeStruct((N, H), jnp.bfloat16),
        ],
    )(acc, acc, hp, dinv, b, m, wn)


def _tc_tail_body(a0_ref, a1_ref, hp_ref, dinv_ref, b_ref, m_ref,
                  batch_ref, wm1_ref, bm1_ref, wm2t_ref, bm2_ref, m4_ref,
                  out_ref, sums_ref, cnt_ref):
    i = pl.program_id(0)
    dinv = dinv_ref[...]
    a = a0_ref[0].astype(jnp.float32) + a1_ref[0].astype(jnp.float32)
    h = (a + hp_ref[...]) * dinv + b_ref[...]
    h3 = jnp.maximum(h, 0.0) * m_ref[...]

    oh_t = (lax.broadcasted_iota(jnp.int32, (_BR, G), 1)
            == batch_ref[...]).astype(jnp.float32)

    @pl.when(i == 0)
    def _():
        sums_ref[...] = jnp.zeros((G, H), jnp.float32)
        cnt_ref[...] = jnp.zeros((G, 128), jnp.float32)

    sums_ref[...] += lax.dot_general(oh_t, h3, (((0,), (0,)), ((), ())),
                                     preferred_element_type=jnp.float32)
    cnt_ref[...] += jnp.sum(oh_t, axis=0)[:, None]

    @pl.when(i == N // _BR - 1)
    def _():
        pooled = sums_ref[...] / jnp.maximum(cnt_ref[:, :1], 1.0)
        z = jnp.maximum(
            jnp.dot(pooled, wm1_ref[...], preferred_element_type=jnp.float32)
            + bm1_ref[...], 0.0) * m4_ref[...]
        out_ref[...] = jnp.sum(z * wm2t_ref[...], axis=1,
                               keepdims=True) + bm2_ref[...]


def _tc_tail(acc, hp, dinv, b, m, batch2, wm1, bm1, wm2t, bm2, m4):
    return pl.pallas_call(
        _tc_tail_body,
        grid=(N // _BR,),
        in_specs=[
            pl.BlockSpec((1, _BR, H), lambda i: (0, i, 0)),
            pl.BlockSpec((1, _BR, H), lambda i: (1, i, 0)),
            pl.BlockSpec((_BR, H), lambda i: (i, 0)),
            pl.BlockSpec((_BR, 1), lambda i: (i, 0)),
            pl.BlockSpec((1, H), lambda i: (0, 0)),
            pl.BlockSpec((_BR, H), lambda i: (i, 0)),
            pl.BlockSpec((_BR, 1), lambda i: (i, 0)),
            pl.BlockSpec((H, H), lambda i: (0, 0)),
            pl.BlockSpec((1, H), lambda i: (0, 0)),
            pl.BlockSpec((1, H), lambda i: (0, 0)),
            pl.BlockSpec((1, 1), lambda i: (0, 0)),
            pl.BlockSpec((G, H), lambda i: (0, 0)),
        ],
        out_specs=pl.BlockSpec((G, 1), lambda i: (0, 0)),
        out_shape=jax.ShapeDtypeStruct((G, 1), jnp.float32),
        scratch_shapes=[
            pltpu.VMEM((G, H), jnp.float32),
            pltpu.VMEM((G, 128), jnp.float32),
        ],
    )(acc, acc, hp, dinv, b, m, batch2, wm1, bm1, wm2t, bm2, m4)


def kernel(x, edge_index, batch, W1, b1, W2, b2, W3, b3, Wm1, bm1, Wm2, bm2):
    src = edge_index[0]
    dst = edge_index[1]

    pad = EPT_PAD - EPT
    src_r = jnp.concatenate(
        [src.reshape(NW, EPT),
         jnp.zeros((NW, pad), jnp.int32)], axis=1).reshape(NW, NJ, KBCH)
    dst_r = jnp.concatenate(
        [dst.reshape(NW, EPT),
         jnp.full((NW, pad), PAD_DST, jnp.int32)], axis=1).reshape(NW, NJ, KBCH)

    ones16 = jnp.ones((KBCH, 16), jnp.float32)
    zeros16 = jnp.zeros((ZROWS, 16), jnp.float32)
    zerosH = jnp.zeros((ZROWS, H), jnp.bfloat16)

    dk = jax.random.split(jax.random.key(42), 4)
    inv_keep = 1.0 / (1.0 - P)
    m1 = jax.random.bernoulli(dk[0], 1.0 - P, (N, H)).astype(jnp.float32) * inv_keep
    m2 = jax.random.bernoulli(dk[1], 1.0 - P, (N, H)).astype(jnp.float32) * inv_keep
    m3 = jax.random.bernoulli(dk[2], 1.0 - P, (N, H)).astype(jnp.float32) * inv_keep
    m4 = jax.random.bernoulli(dk[3], 1.0 - P, (G, H)).astype(jnp.float32) * inv_keep

    deg_out = _sc_deg(dst_r, ones16, zeros16)
    hp1, hp1b, dinv = _tc_pre(x, W1, deg_out)

    acc1 = _sc_edge(hp1b, src_r, dst_r, zerosH)
    hp2, hp2b = _tc_epi(acc1, hp1, dinv, b1.reshape(1, H), m1, W2)

    acc2 = _sc_edge(hp2b, src_r, dst_r, zerosH)
    hp3, hp3b = _tc_epi(acc2, hp2, dinv, b2.reshape(1, H), m2, W3)

    acc3 = _sc_edge(hp3b, src_r, dst_r, zerosH)
    return _tc_tail(acc3, hp3, dinv, b3.reshape(1, H), m3,
                    batch.reshape(N, 1), Wm1, bm1.reshape(1, H),
                    Wm2.reshape(1, H), bm2.reshape(1, 1), m4)


# speedup vs baseline: 1.0099x; 1.0099x over previous
import functools

import jax
import jax.numpy as jnp
import numpy as np
from jax import lax
from jax.experimental import pallas as pl
from jax.experimental.pallas import tpu as pltpu
from jax.experimental.pallas import tpu_sc as plsc

N = 10000
E = 320000
DF = 128
H = 64
G = 64
P = 0.3

NC = 2
NS = 16
NW = NC * NS

CH = 128
EPT = E // NW
KB = 10
KCH = 80
KBCH = KB * CH
EPT_PAD = KCH * CH
NJ = EPT_PAD // KBCH
ACC_ROWS = 10240
ZROWS = ACC_ROWS // NS
PAD_DST = N

_mesh = plsc.VectorSubcoreMesh(core_axis_name="c", subcore_axis_name="s")
_sc_params = pltpu.CompilerParams(use_tc_tiling_on_sc=False)


def _dropout_masks():
    dk = jax.random.split(jax.random.key(42), 4)
    inv_keep = np.float32(1.0) / np.float32(1.0 - P)
    out = []
    for k, shape in zip(dk, [(N, H), (N, H), (N, H), (G, H)]):
        keep = jax.random.bernoulli(k, 1.0 - P, shape)
        out.append(np.asarray(keep, np.float32) * inv_keep)
    return out


_M1, _M2, _M3, _M4 = _dropout_masks()


@functools.partial(
    pl.kernel,
    out_type=jax.ShapeDtypeStruct((NC, ACC_ROWS, 16), jnp.float32),
    mesh=_mesh,
    scratch_types=[
        pltpu.VMEM((NJ, KBCH), jnp.int32),
        pltpu.VMEM((KBCH, 16), jnp.float32),
        pltpu.VMEM_SHARED((ACC_ROWS, 16), jnp.float32),
    ],
    compiler_params=_sc_params,
)
def _sc_deg(dst_hbm, ones_hbm, zeros16_hbm, out_hbm, idx_v, ones_v, acc):
    c = lax.axis_index("c")
    s = lax.axis_index("s")
    wid = c * NS + s
    pltpu.sync_copy(zeros16_hbm, acc.at[pl.ds(s * ZROWS, ZROWS)])
    pltpu.sync_copy(ones_hbm, ones_v)
    pltpu.sync_copy(dst_hbm.at[wid], idx_v)
    plsc.subcore_barrier()

    for j in range(NJ):
        pltpu.sync_copy(ones_v, acc.at[idx_v.at[j]], add=True)
    plsc.subcore_barrier()
    pltpu.sync_copy(acc.at[pl.ds(s * ZROWS, ZROWS)],
                    out_hbm.at[c, pl.ds(s * ZROWS, ZROWS)])


@functools.partial(
    pl.kernel,
    out_type=jax.ShapeDtypeStruct((NC, ACC_ROWS, H), jnp.bfloat16),
    mesh=_mesh,
    scratch_types=[
        pltpu.VMEM((NJ, KBCH), jnp.int32),
        pltpu.VMEM((NJ, KBCH), jnp.int32),
        pltpu.VMEM((2, KBCH, H), jnp.bfloat16),
        pltpu.VMEM_SHARED((ACC_ROWS, H), jnp.bfloat16),
        pltpu.SemaphoreType.DMA((2,)),
        pltpu.SemaphoreType.DMA((2,)),
    ],
    compiler_params=_sc_params,
)
def _sc_edge(hp_hbm, src_hbm, dst_hbm, zeros_hbm, out_hbm,
             sidx_v, didx_v, rows_v, acc, gsem, ssem):
    c = lax.axis_index("c")
    s = lax.axis_index("s")
    wid = c * NS + s
    pltpu.sync_copy(zeros_hbm, acc.at[pl.ds(s * ZROWS, ZROWS)])
    pltpu.sync_copy(src_hbm.at[wid], sidx_v)
    pltpu.sync_copy(dst_hbm.at[wid], didx_v)
    plsc.subcore_barrier()

    pltpu.async_copy(hp_hbm.at[sidx_v.at[0]], rows_v.at[0], gsem.at[0])
    for j in range(NJ):
        b = j % 2
        pltpu.make_async_copy(hp_hbm.at[pl.ds(0, KBCH)], rows_v.at[b],
                              gsem.at[b]).wait()
        if j + 1 < NJ:
            if j >= 1:
                pltpu.make_async_copy(rows_v.at[1 - b], acc.at[pl.ds(0, KBCH)],
                                      ssem.at[1 - b]).wait()
            pltpu.async_copy(hp_hbm.at[sidx_v.at[j + 1]], rows_v.at[1 - b],
                             gsem.at[1 - b])
        pltpu.async_copy(rows_v.at[b], acc.at[didx_v.at[j]], ssem.at[b],
                         add=True)
    pltpu.make_async_copy(rows_v.at[0], acc.at[pl.ds(0, KBCH)],
                          ssem.at[0]).wait()
    pltpu.make_async_copy(rows_v.at[1], acc.at[pl.ds(0, KBCH)],
                          ssem.at[1]).wait()
    plsc.subcore_barrier()
    pltpu.sync_copy(acc.at[pl.ds(s * ZROWS, ZROWS)],
                    out_hbm.at[c, pl.ds(s * ZROWS, ZROWS)])


_BR = 1000


def _tc_pre_body(x_ref, w_ref, d0_ref, d1_ref, hp_ref, hpb_ref, dinv_ref):
    deg = d0_ref[0, :, 0] + d1_ref[0, :, 0] + 1.0
    dinv = lax.rsqrt(deg)
    h = jnp.dot(x_ref[...], w_ref[...], preferred_element_type=jnp.float32)
    hp = h * dinv[:, None]
    hp_ref[...] = hp
    hpb_ref[...] = hp.astype(jnp.bfloat16)
    dinv_ref[...] = dinv[:, None]


def _tc_pre(x, w1, deg_out):
    return pl.pallas_call(
        _tc_pre_body,
        grid=(N // _BR,),
        in_specs=[
            pl.BlockSpec((_BR, DF), lambda i: (i, 0)),
            pl.BlockSpec((DF, H), lambda i: (0, 0)),
            pl.BlockSpec((1, _BR, 16), lambda i: (0, i, 0)),
            pl.BlockSpec((1, _BR, 16), lambda i: (1, i, 0)),
        ],
        out_specs=[
            pl.BlockSpec((_BR, H), lambda i: (i, 0)),
            pl.BlockSpec((_BR, H), lambda i: (i, 0)),
            pl.BlockSpec((_BR, 1), lambda i: (i, 0)),
        ],
        out_shape=[
            jax.ShapeDtypeStruct((N, H), jnp.float32),
            jax.ShapeDtypeStruct((N, H), jnp.bfloat16),
            jax.ShapeDtypeStruct((N, 1), jnp.float32),
        ],
    )(x, w1, deg_out, deg_out)


def _tc_epi_body(a0_ref, a1_ref, hp_ref, dinv_ref, b_ref, m_ref, w_ref,
                 hpn_ref, hpnb_ref):
    dinv = dinv_ref[...]
    a = a0_ref[0].astype(jnp.float32) + a1_ref[0].astype(jnp.float32)
    h = (a + hp_ref[...]) * dinv + b_ref[...]
    h = jnp.maximum(h, 0.0) * m_ref[...]
    hpn = jnp.dot(h, w_ref[...], preferred_element_type=jnp.float32) * dinv
    hpn_ref[...] = hpn
    hpnb_ref[...] = hpn.astype(jnp.bfloat16)


def _tc_epi(acc, hp, dinv, b, m, wn):
    return pl.pallas_call(
        _tc_epi_body,
        grid=(N // _BR,),
        in_specs=[
            pl.BlockSpec((1, _BR, H), lambda i: (0, i, 0)),
            pl.BlockSpec((1, _BR, H), lambda i: (1, i, 0)),
            pl.BlockSpec((_BR, H), lambda i: (i, 0)),
            pl.BlockSpec((_BR, 1), lambda i: (i, 0)),
            pl.BlockSpec((1, H), lambda i: (0, 0)),
            pl.BlockSpec((_BR, H), lambda i: (i, 0)),
            pl.BlockSpec((H, H), lambda i: (0, 0)),
        ],
        out_specs=[
            pl.BlockSpec((_BR, H), lambda i: (i, 0)),
            pl.BlockSpec((_BR, H), lambda i: (i, 0)),
        ],
        out_shape=[
            jax.ShapeDtypeStruct((N, H), jnp.float32),
            jax.ShapeDtypeStruct((N, H), jnp.bfloat16),
        ],
    )(acc, acc, hp, dinv, b, m, wn)


def _tc_tail_body(a0_ref, a1_ref, hp_ref, dinv_ref, b_ref, m_ref,
                  batch_ref, wm1_ref, bm1_ref, wm2t_ref, bm2_ref, m4_ref,
                  out_ref, sums_ref, cnt_ref):
    i = pl.program_id(0)
    dinv = dinv_ref[...]
    a = a0_ref[0].astype(jnp.float32) + a1_ref[0].astype(jnp.float32)
    h = (a + hp_ref[...]) * dinv + b_ref[...]
    h3 = jnp.maximum(h, 0.0) * m_ref[...]

    oh_t = (lax.broadcasted_iota(jnp.int32, (_BR, G), 1)
            == batch_ref[...]).astype(jnp.float32)

    @pl.when(i == 0)
    def _():
        sums_ref[...] = jnp.zeros((G, H), jnp.float32)
        cnt_ref[...] = jnp.zeros((G, 128), jnp.float32)

    sums_ref[...] += lax.dot_general(oh_t, h3, (((0,), (0,)), ((), ())),
                                     preferred_element_type=jnp.float32)
    cnt_ref[...] += jnp.sum(oh_t, axis=0)[:, None]

    @pl.when(i == N // _BR - 1)
    def _():
        pooled = sums_ref[...] / jnp.maximum(cnt_ref[:, :1], 1.0)
        z = jnp.maximum(
            jnp.dot(pooled, wm1_ref[...], preferred_element_type=jnp.float32)
            + bm1_ref[...], 0.0) * m4_ref[...]
        out_ref[...] = jnp.sum(z * wm2t_ref[...], axis=1,
                               keepdims=True) + bm2_ref[...]


def _tc_tail(acc, hp, dinv, b, m, batch2, wm1, bm1, wm2t, bm2, m4):
    return pl.pallas_call(
        _tc_tail_body,
        grid=(N // _BR,),
        in_specs=[
            pl.BlockSpec((1, _BR, H), lambda i: (0, i, 0)),
            pl.BlockSpec((1, _BR, H), lambda i: (1, i, 0)),
            pl.BlockSpec((_BR, H), lambda i: (i, 0)),
            pl.BlockSpec((_BR, 1), lambda i: (i, 0)),
            pl.BlockSpec((1, H), lambda i: (0, 0)),
            pl.BlockSpec((_BR, H), lambda i: (i, 0)),
            pl.BlockSpec((_BR, 1), lambda i: (i, 0)),
            pl.BlockSpec((H, H), lambda i: (0, 0)),
            pl.BlockSpec((1, H), lambda i: (0, 0)),
            pl.BlockSpec((1, H), lambda i: (0, 0)),
            pl.BlockSpec((1, 1), lambda i: (0, 0)),
            pl.BlockSpec((G, H), lambda i: (0, 0)),
        ],
        out_specs=pl.BlockSpec((G, 1), lambda i: (0, 0)),
        out_shape=jax.ShapeDtypeStruct((G, 1), jnp.float32),
        scratch_shapes=[
            pltpu.VMEM((G, H), jnp.float32),
            pltpu.VMEM((G, 128), jnp.float32),
        ],
    )(acc, acc, hp, dinv, b, m, batch2, wm1, bm1, wm2t, bm2, m4)


def kernel(x, edge_index, batch, W1, b1, W2, b2, W3, b3, Wm1, bm1, Wm2, bm2):
    src = edge_index[0]
    dst = edge_index[1]

    pad = EPT_PAD - EPT
    src_r = jnp.concatenate(
        [src.reshape(NW, EPT),
         jnp.zeros((NW, pad), jnp.int32)], axis=1).reshape(NW, NJ, KBCH)
    dst_r = jnp.concatenate(
        [dst.reshape(NW, EPT),
         jnp.full((NW, pad), PAD_DST, jnp.int32)], axis=1).reshape(NW, NJ, KBCH)

    ones16 = jnp.ones((KBCH, 16), jnp.float32)
    zeros16 = jnp.zeros((ZROWS, 16), jnp.float32)
    zerosH = jnp.zeros((ZROWS, H), jnp.bfloat16)

    m1 = jnp.asarray(_M1)
    m2 = jnp.asarray(_M2)
    m3 = jnp.asarray(_M3)
    m4 = jnp.asarray(_M4)

    deg_out = _sc_deg(dst_r, ones16, zeros16)
    hp1, hp1b, dinv = _tc_pre(x, W1, deg_out)

    acc1 = _sc_edge(hp1b, src_r, dst_r, zerosH)
    hp2, hp2b = _tc_epi(acc1, hp1, dinv, b1.reshape(1, H), m1, W2)

    acc2 = _sc_edge(hp2b, src_r, dst_r, zerosH)
    hp3, hp3b = _tc_epi(acc2, hp2, dinv, b2.reshape(1, H), m2, W3)

    acc3 = _sc_edge(hp3b, src_r, dst_r, zerosH)
    return _tc_tail(acc3, hp3, dinv, b3.reshape(1, H), m3,
                    batch.reshape(N, 1), Wm1, bm1.reshape(1, H),
                    Wm2.reshape(1, H), bm2.reshape(1, 1), m4)


# speedup vs baseline: 1.0100x; 1.0001x over previous
import functools

import jax
import jax.numpy as jnp
import numpy as np
from jax import lax
from jax.experimental import pallas as pl
from jax.experimental.pallas import tpu as pltpu
from jax.experimental.pallas import tpu_sc as plsc

N = 10000
E = 320000
DF = 128
H = 64
G = 64
P = 0.3

NC = 2
NS = 16
NW = NC * NS

CH = 128
EPT = E // NW
KB = 10
KCH = 80
KBCH = KB * CH
EPT_PAD = KCH * CH
NJ = EPT_PAD // KBCH
ACC_ROWS = 10240
ZROWS = ACC_ROWS // NS
PAD_DST = N

_mesh = plsc.VectorSubcoreMesh(core_axis_name="c", subcore_axis_name="s")
_sc_params = pltpu.CompilerParams(use_tc_tiling_on_sc=False)


def _dropout_masks():
    dk = jax.random.split(jax.random.key(42), 4)
    inv_keep = np.float32(1.0) / np.float32(1.0 - P)
    out = []
    for k, shape in zip(dk, [(N, H), (N, H), (N, H), (G, H)]):
        keep = jax.random.bernoulli(k, 1.0 - P, shape)
        out.append(np.asarray(keep, np.float32) * inv_keep)
    return out


_M1, _M2, _M3, _M4 = _dropout_masks()


@functools.partial(
    pl.kernel,
    out_type=jax.ShapeDtypeStruct((NC, ACC_ROWS, 16), jnp.float32),
    mesh=_mesh,
    scratch_types=[
        pltpu.VMEM((NJ, KBCH), jnp.int32),
        pltpu.VMEM((KBCH, 16), jnp.float32),
        pltpu.VMEM_SHARED((ACC_ROWS, 16), jnp.float32),
    ],
    compiler_params=_sc_params,
)
def _sc_deg(dst_hbm, ones_hbm, zeros16_hbm, out_hbm, idx_v, ones_v, acc):
    c = lax.axis_index("c")
    s = lax.axis_index("s")
    wid = c * NS + s
    pltpu.sync_copy(zeros16_hbm, acc.at[pl.ds(s * ZROWS, ZROWS)])
    pltpu.sync_copy(ones_hbm, ones_v)
    pltpu.sync_copy(dst_hbm.at[wid], idx_v)
    plsc.subcore_barrier()

    for j in range(NJ):
        pltpu.sync_copy(ones_v, acc.at[idx_v.at[j]], add=True)
    plsc.subcore_barrier()
    pltpu.sync_copy(acc.at[pl.ds(s * ZROWS, ZROWS)],
                    out_hbm.at[c, pl.ds(s * ZROWS, ZROWS)])


@functools.partial(
    pl.kernel,
    out_type=jax.ShapeDtypeStruct((NC, ACC_ROWS, H), jnp.bfloat16),
    mesh=_mesh,
    scratch_types=[
        pltpu.VMEM((NJ, KBCH), jnp.int32),
        pltpu.VMEM((NJ, KBCH), jnp.int32),
        pltpu.VMEM((2, KBCH, H), jnp.bfloat16),
        pltpu.VMEM_SHARED((ACC_ROWS, H), jnp.bfloat16),
        pltpu.SemaphoreType.DMA((2,)),
        pltpu.SemaphoreType.DMA((2,)),
    ],
    compiler_params=_sc_params,
)
def _sc_edge(hp_hbm, src_hbm, dst_hbm, zeros_hbm, out_hbm,
             sidx_v, didx_v, rows_v, acc, gsem, ssem):
    c = lax.axis_index("c")
    s = lax.axis_index("s")
    wid = c * NS + s
    pltpu.sync_copy(zeros_hbm, acc.at[pl.ds(s * ZROWS, ZROWS)])
    pltpu.sync_copy(src_hbm.at[wid], sidx_v)
    pltpu.sync_copy(dst_hbm.at[wid], didx_v)
    plsc.subcore_barrier()

    pltpu.async_copy(hp_hbm.at[sidx_v.at[0]], rows_v.at[0], gsem.at[0])
    for j in range(NJ):
        b = j % 2
        pltpu.make_async_copy(hp_hbm.at[pl.ds(0, KBCH)], rows_v.at[b],
                              gsem.at[b]).wait()
        if j + 1 < NJ:
            if j >= 1:
                pltpu.make_async_copy(rows_v.at[1 - b], acc.at[pl.ds(0, KBCH)],
                                      ssem.at[1 - b]).wait()
            pltpu.async_copy(hp_hbm.at[sidx_v.at[j + 1]], rows_v.at[1 - b],
                             gsem.at[1 - b])
        pltpu.async_copy(rows_v.at[b], acc.at[didx_v.at[j]], ssem.at[b],
                         add=True)
    pltpu.make_async_copy(rows_v.at[0], acc.at[pl.ds(0, KBCH)],
                          ssem.at[0]).wait()
    pltpu.make_async_copy(rows_v.at[1], acc.at[pl.ds(0, KBCH)],
                          ssem.at[1]).wait()
    plsc.subcore_barrier()
    pltpu.sync_copy(acc.at[pl.ds(s * ZROWS, ZROWS)],
                    out_hbm.at[c, pl.ds(s * ZROWS, ZROWS)])


_BR = 1000


def _tc_mm1_body(x_ref, w_ref, h_ref):
    h_ref[...] = jnp.dot(x_ref[...], w_ref[...],
                         preferred_element_type=jnp.float32)


def _tc_mm1(x, w1):
    return pl.pallas_call(
        _tc_mm1_body,
        grid=(N // _BR,),
        in_specs=[
            pl.BlockSpec((_BR, DF), lambda i: (i, 0)),
            pl.BlockSpec((DF, H), lambda i: (0, 0)),
        ],
        out_specs=pl.BlockSpec((_BR, H), lambda i: (i, 0)),
        out_shape=jax.ShapeDtypeStruct((N, H), jnp.float32),
    )(x, w1)


def _tc_pre_body(h_ref, d0_ref, d1_ref, hp_ref, hpb_ref, dinv_ref):
    deg = d0_ref[0, :, 0] + d1_ref[0, :, 0] + 1.0
    dinv = lax.rsqrt(deg)
    hp = h_ref[...] * dinv[:, None]
    hp_ref[...] = hp
    hpb_ref[...] = hp.astype(jnp.bfloat16)
    dinv_ref[...] = dinv[:, None]


def _tc_pre(h, deg_out):
    return pl.pallas_call(
        _tc_pre_body,
        grid=(N // _BR,),
        in_specs=[
            pl.BlockSpec((_BR, H), lambda i: (i, 0)),
            pl.BlockSpec((1, _BR, 16), lambda i: (0, i, 0)),
            pl.BlockSpec((1, _BR, 16), lambda i: (1, i, 0)),
        ],
        out_specs=[
            pl.BlockSpec((_BR, H), lambda i: (i, 0)),
            pl.BlockSpec((_BR, H), lambda i: (i, 0)),
            pl.BlockSpec((_BR, 1), lambda i: (i, 0)),
        ],
        out_shape=[
            jax.ShapeDtypeStruct((N, H), jnp.float32),
            jax.ShapeDtypeStruct((N, H), jnp.bfloat16),
            jax.ShapeDtypeStruct((N, 1), jnp.float32),
        ],
    )(h, deg_out, deg_out)


def _tc_epi_body(a0_ref, a1_ref, hp_ref, dinv_ref, b_ref, m_ref, w_ref,
                 hpn_ref, hpnb_ref):
    dinv = dinv_ref[...]
    a = a0_ref[0].astype(jnp.float32) + a1_ref[0].astype(jnp.float32)
    h = (a + hp_ref[...]) * dinv + b_ref[...]
    h = jnp.maximum(h, 0.0) * m_ref[...]
    hpn = jnp.dot(h, w_ref[...], preferred_element_type=jnp.float32) * dinv
    hpn_ref[...] = hpn
    hpnb_ref[...] = hpn.astype(jnp.bfloat16)


def _tc_epi(acc, hp, dinv, b, m, wn):
    return pl.pallas_call(
        _tc_epi_body,
        grid=(N // _BR,),
        in_specs=[
            pl.BlockSpec((1, _BR, H), lambda i: (0, i, 0)),
            pl.BlockSpec((1, _BR, H), lambda i: (1, i, 0)),
            pl.BlockSpec((_BR, H), lambda i: (i, 0)),
            pl.BlockSpec((_BR, 1), lambda i: (i, 0)),
            pl.BlockSpec((1, H), lambda i: (0, 0)),
            pl.BlockSpec((_BR, H), lambda i: (i, 0)),
            pl.BlockSpec((H, H), lambda i: (0, 0)),
        ],
        out_specs=[
            pl.BlockSpec((_BR, H), lambda i: (i, 0)),
            pl.BlockSpec((_BR, H), lambda i: (i, 0)),
        ],
        out_shape=[
            jax.ShapeDtypeStruct((N, H), jnp.float32),
            jax.ShapeDtypeStruct((N, H), jnp.bfloat16),
        ],
    )(acc, acc, hp, dinv, b, m, wn)


def _tc_tail_body(a0_ref, a1_ref, hp_ref, dinv_ref, b_ref, m_ref,
                  batch_ref, wm1_ref, bm1_ref, wm2t_ref, bm2_ref, m4_ref,
                  out_ref, sums_ref, cnt_ref):
    i = pl.program_id(0)
    dinv = dinv_ref[...]
    a = a0_ref[0].astype(jnp.float32) + a1_ref[0].astype(jnp.float32)
    h = (a + hp_ref[...]) * dinv + b_ref[...]
    h3 = jnp.maximum(h, 0.0) * m_ref[...]

    oh_t = (lax.broadcasted_iota(jnp.int32, (_BR, G), 1)
            == batch_ref[...]).astype(jnp.float32)

    @pl.when(i == 0)
    def _():
        sums_ref[...] = jnp.zeros((G, H), jnp.float32)
        cnt_ref[...] = jnp.zeros((G, 128), jnp.float32)

    sums_ref[...] += lax.dot_general(oh_t, h3, (((0,), (0,)), ((), ())),
                                     preferred_element_type=jnp.float32)
    cnt_ref[...] += jnp.sum(oh_t, axis=0)[:, None]

    @pl.when(i == N // _BR - 1)
    def _():
        pooled = sums_ref[...] / jnp.maximum(cnt_ref[:, :1], 1.0)
        z = jnp.maximum(
            jnp.dot(pooled, wm1_ref[...], preferred_element_type=jnp.float32)
            + bm1_ref[...], 0.0) * m4_ref[...]
        out_ref[...] = jnp.sum(z * wm2t_ref[...], axis=1,
                               keepdims=True) + bm2_ref[...]


def _tc_tail(acc, hp, dinv, b, m, batch2, wm1, bm1, wm2t, bm2, m4):
    return pl.pallas_call(
        _tc_tail_body,
        grid=(N // _BR,),
        in_specs=[
            pl.BlockSpec((1, _BR, H), lambda i: (0, i, 0)),
            pl.BlockSpec((1, _BR, H), lambda i: (1, i, 0)),
            pl.BlockSpec((_BR, H), lambda i: (i, 0)),
            pl.BlockSpec((_BR, 1), lambda i: (i, 0)),
            pl.BlockSpec((1, H), lambda i: (0, 0)),
            pl.BlockSpec((_BR, H), lambda i: (i, 0)),
            pl.BlockSpec((_BR, 1), lambda i: (i, 0)),
            pl.BlockSpec((H, H), lambda i: (0, 0)),
            pl.BlockSpec((1, H), lambda i: (0, 0)),
            pl.BlockSpec((1, H), lambda i: (0, 0)),
            pl.BlockSpec((1, 1), lambda i: (0, 0)),
            pl.BlockSpec((G, H), lambda i: (0, 0)),
        ],
        out_specs=pl.BlockSpec((G, 1), lambda i: (0, 0)),
        out_shape=jax.ShapeDtypeStruct((G, 1), jnp.float32),
        scratch_shapes=[
            pltpu.VMEM((G, H), jnp.float32),
            pltpu.VMEM((G, 128), jnp.float32),
        ],
    )(acc, acc, hp, dinv, b, m, batch2, wm1, bm1, wm2t, bm2, m4)


def kernel(x, edge_index, batch, W1, b1, W2, b2, W3, b3, Wm1, bm1, Wm2, bm2):
    src = edge_index[0]
    dst = edge_index[1]

    pad = EPT_PAD - EPT
    src_r = jnp.concatenate(
        [src.reshape(NW, EPT),
         jnp.zeros((NW, pad), jnp.int32)], axis=1).reshape(NW, NJ, KBCH)
    dst_r = jnp.concatenate(
        [dst.reshape(NW, EPT),
         jnp.full((NW, pad), PAD_DST, jnp.int32)], axis=1).reshape(NW, NJ, KBCH)

    ones16 = jnp.ones((KBCH, 16), jnp.float32)
    zeros16 = jnp.zeros((ZROWS, 16), jnp.float32)
    zerosH = jnp.zeros((ZROWS, H), jnp.bfloat16)

    m1 = jnp.asarray(_M1)
    m2 = jnp.asarray(_M2)
    m3 = jnp.asarray(_M3)
    m4 = jnp.asarray(_M4)

    deg_out = _sc_deg(dst_r, ones16, zeros16)
    h1u = _tc_mm1(x, W1)
    hp1, hp1b, dinv = _tc_pre(h1u, deg_out)

    acc1 = _sc_edge(hp1b, src_r, dst_r, zerosH)
    hp2, hp2b = _tc_epi(acc1, hp1, dinv, b1.reshape(1, H), m1, W2)

    acc2 = _sc_edge(hp2b, src_r, dst_r, zerosH)
    hp3, hp3b = _tc_epi(acc2, hp2, dinv, b2.reshape(1, H), m2, W3)

    acc3 = _sc_edge(hp3b, src_r, dst_r, zerosH)
    return _tc_tail(acc3, hp3, dinv, b3.reshape(1, H), m3,
                    batch.reshape(N, 1), Wm1, bm1.reshape(1, H),
                    Wm2.reshape(1, H), bm2.reshape(1, 1), m4)


# speedup vs baseline: 1.0223x; 1.0122x over previous
import functools

import jax
import jax.numpy as jnp
import numpy as np
from jax import lax
from jax.experimental import pallas as pl
from jax.experimental.pallas import tpu as pltpu
from jax.experimental.pallas import tpu_sc as plsc

N = 10000
E = 320000
DF = 128
H = 64
G = 64
P = 0.3

NC = 2
NS = 16
NW = NC * NS

CH = 128
EPT = E // NW
KB = 10
KCH = 80
KBCH = KB * CH
EPT_PAD = KCH * CH
NJ = EPT_PAD // KBCH
ACC_ROWS = 10240
ZROWS = ACC_ROWS // NS
PAD_DST = N

_mesh = plsc.VectorSubcoreMesh(core_axis_name="c", subcore_axis_name="s")
_sc_params = pltpu.CompilerParams(use_tc_tiling_on_sc=False)


def _dropout_masks():
    dk = jax.random.split(jax.random.key(42), 4)
    inv_keep = np.float32(1.0) / np.float32(1.0 - P)
    out = []
    for k, shape in zip(dk, [(N, H), (N, H), (N, H), (G, H)]):
        keep = jax.random.bernoulli(k, 1.0 - P, shape)
        out.append(np.asarray(keep, np.float32) * inv_keep)
    return out


_M1, _M2, _M3, _M4 = _dropout_masks()


@functools.partial(
    pl.kernel,
    out_type=jax.ShapeDtypeStruct((NC, ACC_ROWS, 16), jnp.float32),
    mesh=_mesh,
    scratch_types=[
        pltpu.VMEM((NJ, KBCH), jnp.int32),
        pltpu.VMEM((KBCH, 16), jnp.float32),
        pltpu.VMEM_SHARED((ACC_ROWS, 16), jnp.float32),
        pltpu.SemaphoreType.DMA,
    ],
    compiler_params=_sc_params,
)
def _sc_deg(dst_hbm, ones_hbm, zeros16_hbm, out_hbm, idx_v, ones_v, acc, sem):
    c = lax.axis_index("c")
    s = lax.axis_index("s")
    wid = c * NS + s
    d1 = pltpu.async_copy(zeros16_hbm, acc.at[pl.ds(s * ZROWS, ZROWS)], sem)
    d2 = pltpu.async_copy(ones_hbm, ones_v, sem)
    d3 = pltpu.async_copy(dst_hbm.at[wid], idx_v, sem)
    d1.wait()
    d2.wait()
    d3.wait()
    plsc.subcore_barrier()

    for j in range(NJ):
        pltpu.sync_copy(ones_v, acc.at[idx_v.at[j]], add=True)
    plsc.subcore_barrier()
    pltpu.sync_copy(acc.at[pl.ds(s * ZROWS, ZROWS)],
                    out_hbm.at[c, pl.ds(s * ZROWS, ZROWS)])


@functools.partial(
    pl.kernel,
    out_type=jax.ShapeDtypeStruct((NC, ACC_ROWS, H), jnp.bfloat16),
    mesh=_mesh,
    scratch_types=[
        pltpu.VMEM((NJ, KBCH), jnp.int32),
        pltpu.VMEM((NJ, KBCH), jnp.int32),
        pltpu.VMEM((2, KBCH, H), jnp.bfloat16),
        pltpu.VMEM_SHARED((ACC_ROWS, H), jnp.bfloat16),
        pltpu.SemaphoreType.DMA((2,)),
        pltpu.SemaphoreType.DMA((2,)),
    ],
    compiler_params=_sc_params,
)
def _sc_edge(hp_hbm, src_hbm, dst_hbm, zeros_hbm, out_hbm,
             sidx_v, didx_v, rows_v, acc, gsem, ssem):
    c = lax.axis_index("c")
    s = lax.axis_index("s")
    wid = c * NS + s
    d1 = pltpu.async_copy(zeros_hbm, acc.at[pl.ds(s * ZROWS, ZROWS)], gsem.at[0])
    d2 = pltpu.async_copy(src_hbm.at[wid], sidx_v, gsem.at[0])
    d3 = pltpu.async_copy(dst_hbm.at[wid], didx_v, gsem.at[0])
    d1.wait()
    d2.wait()
    d3.wait()
    plsc.subcore_barrier()

    pltpu.async_copy(hp_hbm.at[sidx_v.at[0]], rows_v.at[0], gsem.at[0])
    for j in range(NJ):
        b = j % 2
        pltpu.make_async_copy(hp_hbm.at[pl.ds(0, KBCH)], rows_v.at[b],
                              gsem.at[b]).wait()
        if j + 1 < NJ:
            if j >= 1:
                pltpu.make_async_copy(rows_v.at[1 - b], acc.at[pl.ds(0, KBCH)],
                                      ssem.at[1 - b]).wait()
            pltpu.async_copy(hp_hbm.at[sidx_v.at[j + 1]], rows_v.at[1 - b],
                             gsem.at[1 - b])
        pltpu.async_copy(rows_v.at[b], acc.at[didx_v.at[j]], ssem.at[b],
                         add=True)
    pltpu.make_async_copy(rows_v.at[0], acc.at[pl.ds(0, KBCH)],
                          ssem.at[0]).wait()
    pltpu.make_async_copy(rows_v.at[1], acc.at[pl.ds(0, KBCH)],
                          ssem.at[1]).wait()
    plsc.subcore_barrier()
    pltpu.sync_copy(acc.at[pl.ds(s * ZROWS, ZROWS)],
                    out_hbm.at[c, pl.ds(s * ZROWS, ZROWS)])


_BR = 1000


def _tc_mm1_body(x_ref, w_ref, h_ref):
    h_ref[...] = jnp.dot(x_ref[...], w_ref[...],
                         preferred_element_type=jnp.float32)


def _tc_mm1(x, w1):
    return pl.pallas_call(
        _tc_mm1_body,
        grid=(N // _BR,),
        in_specs=[
            pl.BlockSpec((_BR, DF), lambda i: (i, 0)),
            pl.BlockSpec((DF, H), lambda i: (0, 0)),
        ],
        out_specs=pl.BlockSpec((_BR, H), lambda i: (i, 0)),
        out_shape=jax.ShapeDtypeStruct((N, H), jnp.float32),
    )(x, w1)


def _tc_pre_body(h_ref, d0_ref, d1_ref, hp_ref, hpb_ref, dinv_ref):
    deg = d0_ref[0, :, 0] + d1_ref[0, :, 0] + 1.0
    dinv = lax.rsqrt(deg)
    hp = h_ref[...] * dinv[:, None]
    hp_ref[...] = hp
    hpb_ref[...] = hp.astype(jnp.bfloat16)
    dinv_ref[...] = dinv[:, None]


def _tc_pre(h, deg_out):
    return pl.pallas_call(
        _tc_pre_body,
        grid=(N // _BR,),
        in_specs=[
            pl.BlockSpec((_BR, H), lambda i: (i, 0)),
            pl.BlockSpec((1, _BR, 16), lambda i: (0, i, 0)),
            pl.BlockSpec((1, _BR, 16), lambda i: (1, i, 0)),
        ],
        out_specs=[
            pl.BlockSpec((_BR, H), lambda i: (i, 0)),
            pl.BlockSpec((_BR, H), lambda i: (i, 0)),
            pl.BlockSpec((_BR, 1), lambda i: (i, 0)),
        ],
        out_shape=[
            jax.ShapeDtypeStruct((N, H), jnp.float32),
            jax.ShapeDtypeStruct((N, H), jnp.bfloat16),
            jax.ShapeDtypeStruct((N, 1), jnp.float32),
        ],
    )(h, deg_out, deg_out)


def _tc_epi_body(a0_ref, a1_ref, hp_ref, dinv_ref, b_ref, m_ref, w_ref,
                 hpn_ref, hpnb_ref):
    dinv = dinv_ref[...]
    a = a0_ref[0].astype(jnp.float32) + a1_ref[0].astype(jnp.float32)
    h = (a + hp_ref[...]) * dinv + b_ref[...]
    h = jnp.maximum(h, 0.0) * m_ref[...]
    hpn = jnp.dot(h, w_ref[...], preferred_element_type=jnp.float32) * dinv
    hpn_ref[...] = hpn
    hpnb_ref[...] = hpn.astype(jnp.bfloat16)


def _tc_epi(acc, hp, dinv, b, m, wn):
    return pl.pallas_call(
        _tc_epi_body,
        grid=(N // _BR,),
        in_specs=[
            pl.BlockSpec((1, _BR, H), lambda i: (0, i, 0)),
            pl.BlockSpec((1, _BR, H), lambda i: (1, i, 0)),
            pl.BlockSpec((_BR, H), lambda i: (i, 0)),
            pl.BlockSpec((_BR, 1), lambda i: (i, 0)),
            pl.BlockSpec((1, H), lambda i: (0, 0)),
            pl.BlockSpec((_BR, H), lambda i: (i, 0)),
            pl.BlockSpec((H, H), lambda i: (0, 0)),
        ],
        out_specs=[
            pl.BlockSpec((_BR, H), lambda i: (i, 0)),
            pl.BlockSpec((_BR, H), lambda i: (i, 0)),
        ],
        out_shape=[
            jax.ShapeDtypeStruct((N, H), jnp.float32),
            jax.ShapeDtypeStruct((N, H), jnp.bfloat16),
        ],
    )(acc, acc, hp, dinv, b, m, wn)


def _tc_tail_body(a0_ref, a1_ref, hp_ref, dinv_ref, b_ref, m_ref,
                  batch_ref, wm1_ref, bm1_ref, wm2t_ref, bm2_ref, m4_ref,
                  out_ref, sums_ref, cnt_ref):
    i = pl.program_id(0)
    dinv = dinv_ref[...]
    a = a0_ref[0].astype(jnp.float32) + a1_ref[0].astype(jnp.float32)
    h = (a + hp_ref[...]) * dinv + b_ref[...]
    h3 = jnp.maximum(h, 0.0) * m_ref[...]

    oh_t = (lax.broadcasted_iota(jnp.int32, (_BR, G), 1)
            == batch_ref[...]).astype(jnp.float32)

    @pl.when(i == 0)
    def _():
        sums_ref[...] = jnp.zeros((G, H), jnp.float32)
        cnt_ref[...] = jnp.zeros((G, 128), jnp.float32)

    sums_ref[...] += lax.dot_general(oh_t, h3, (((0,), (0,)), ((), ())),
                                     preferred_element_type=jnp.float32)
    cnt_ref[...] += jnp.sum(oh_t, axis=0)[:, None]

    @pl.when(i == N // _BR - 1)
    def _():
        pooled = sums_ref[...] / jnp.maximum(cnt_ref[:, :1], 1.0)
        z = jnp.maximum(
            jnp.dot(pooled, wm1_ref[...], preferred_element_type=jnp.float32)
            + bm1_ref[...], 0.0) * m4_ref[...]
        out_ref[...] = jnp.sum(z * wm2t_ref[...], axis=1,
                               keepdims=True) + bm2_ref[...]


def _tc_tail(acc, hp, dinv, b, m, batch2, wm1, bm1, wm2t, bm2, m4):
    return pl.pallas_call(
        _tc_tail_body,
        grid=(N // _BR,),
        in_specs=[
            pl.BlockSpec((1, _BR, H), lambda i: (0, i, 0)),
            pl.BlockSpec((1, _BR, H), lambda i: (1, i, 0)),
            pl.BlockSpec((_BR, H), lambda i: (i, 0)),
            pl.BlockSpec((_BR, 1), lambda i: (i, 0)),
            pl.BlockSpec((1, H), lambda i: (0, 0)),
            pl.BlockSpec((_BR, H), lambda i: (i, 0)),
            pl.BlockSpec((_BR, 1), lambda i: (i, 0)),
            pl.BlockSpec((H, H), lambda i: (0, 0)),
            pl.BlockSpec((1, H), lambda i: (0, 0)),
            pl.BlockSpec((1, H), lambda i: (0, 0)),
            pl.BlockSpec((1, 1), lambda i: (0, 0)),
            pl.BlockSpec((G, H), lambda i: (0, 0)),
        ],
        out_specs=pl.BlockSpec((G, 1), lambda i: (0, 0)),
        out_shape=jax.ShapeDtypeStruct((G, 1), jnp.float32),
        scratch_shapes=[
            pltpu.VMEM((G, H), jnp.float32),
            pltpu.VMEM((G, 128), jnp.float32),
        ],
    )(acc, acc, hp, dinv, b, m, batch2, wm1, bm1, wm2t, bm2, m4)


def kernel(x, edge_index, batch, W1, b1, W2, b2, W3, b3, Wm1, bm1, Wm2, bm2):
    src = edge_index[0]
    dst = edge_index[1]

    pad = EPT_PAD - EPT
    src_r = jnp.concatenate(
        [src.reshape(NW, EPT),
         jnp.zeros((NW, pad), jnp.int32)], axis=1).reshape(NW, NJ, KBCH)
    dst_r = jnp.concatenate(
        [dst.reshape(NW, EPT),
         jnp.full((NW, pad), PAD_DST, jnp.int32)], axis=1).reshape(NW, NJ, KBCH)

    ones16 = jnp.ones((KBCH, 16), jnp.float32)
    zeros16 = jnp.zeros((ZROWS, 16), jnp.float32)
    zerosH = jnp.zeros((ZROWS, H), jnp.bfloat16)

    m1 = jnp.asarray(_M1)
    m2 = jnp.asarray(_M2)
    m3 = jnp.asarray(_M3)
    m4 = jnp.asarray(_M4)

    deg_out = _sc_deg(dst_r, ones16, zeros16)
    h1u = _tc_mm1(x, W1)
    hp1, hp1b, dinv = _tc_pre(h1u, deg_out)

    acc1 = _sc_edge(hp1b, src_r, dst_r, zerosH)
    hp2, hp2b = _tc_epi(acc1, hp1, dinv, b1.reshape(1, H), m1, W2)

    acc2 = _sc_edge(hp2b, src_r, dst_r, zerosH)
    hp3, hp3b = _tc_epi(acc2, hp2, dinv, b2.reshape(1, H), m2, W3)

    acc3 = _sc_edge(hp3b, src_r, dst_r, zerosH)
    return _tc_tail(acc3, hp3, dinv, b3.reshape(1, H), m3,
                    batch.reshape(N, 1), Wm1, bm1.reshape(1, H),
                    Wm2.reshape(1, H), bm2.reshape(1, 1), m4)
